# baseline (speedup 1.0000x reference)
import functools

import jax
import jax.numpy as jnp
from jax import lax
from jax.experimental import pallas as pl
from jax.experimental.pallas import tpu as pltpu
from jax.experimental.pallas import tpu_sc as plsc

N_FACT = 32
B = 4096
NC = 2
NS = 16
NW = NC * NS
BPW = B // NW
L = 16
GROUPS = BPW // L


def _mf_body(users_h, items_h, ue_h, ie_h, ub_h, ib_h, bias_h, out_h,
             idx_u, idx_i, ue_rows, ie_rows, bu_v, bi_v, bias_v, out_v, sem):
    wid = lax.axis_index("s") * NC + lax.axis_index("c")
    base = wid * BPW

    pltpu.sync_copy(users_h.at[pl.ds(base, BPW)], idx_u)
    pltpu.sync_copy(items_h.at[pl.ds(base, BPW)], idx_i)
    pltpu.sync_copy(bias_h, bias_v)

    cps = [
        pltpu.async_copy(ue_h.at[idx_u], ue_rows, sem),
        pltpu.async_copy(ie_h.at[idx_i], ie_rows, sem),
        pltpu.async_copy(ub_h.at[idx_u], bu_v, sem),
        pltpu.async_copy(ib_h.at[idx_i], bi_v, sem),
    ]
    for cp in cps:
        cp.wait()

    bias_vec = bias_v[...]
    lane = lax.broadcasted_iota(jnp.int32, (L,), 0)
    lane_masks = [lane == j for j in range(L)]
    for g in range(GROUPS):
        r0 = g * L
        acc = jnp.zeros((L,), jnp.float32)
        for j in range(L):
            b = r0 + j
            u0 = ue_rows[b, pl.ds(0, L)]
            u1 = ue_rows[b, pl.ds(L, L)]
            i0 = ie_rows[b, pl.ds(0, L)]
            i1 = ie_rows[b, pl.ds(L, L)]
            p = u0 * i0 + u1 * i1
            s = jnp.sum(p)
            acc = jnp.where(lane_masks[j], jnp.full((L,), s), acc)
        acc = acc + bu_v[pl.ds(r0, L)] + bi_v[pl.ds(r0, L)] + bias_vec
        acc = jnp.minimum(jnp.maximum(acc, 1.0), 5.0)
        out_v[pl.ds(r0, L)] = acc

    pltpu.sync_copy(out_v, out_h.at[pl.ds(base, BPW)])


_mf = pl.kernel(
    _mf_body,
    out_type=jax.ShapeDtypeStruct((B,), jnp.float32),
    mesh=plsc.VectorSubcoreMesh(core_axis_name="c", subcore_axis_name="s"),
    compiler_params=pltpu.CompilerParams(needs_layout_passes=False,
                                         use_tc_tiling_on_sc=False),
    scratch_types=[
        pltpu.VMEM((BPW,), jnp.int32),
        pltpu.VMEM((BPW,), jnp.int32),
        pltpu.VMEM((BPW, N_FACT), jnp.float32),
        pltpu.VMEM((BPW, N_FACT), jnp.float32),
        pltpu.VMEM((BPW,), jnp.float32),
        pltpu.VMEM((BPW,), jnp.float32),
        pltpu.VMEM((L,), jnp.float32),
        pltpu.VMEM((BPW,), jnp.float32),
        pltpu.SemaphoreType.DMA,
    ],
)


def kernel(users, items, user_embeddings, item_embeddings, user_biases,
           item_biases, bias):
    ub = user_biases.reshape(-1)
    ib = item_biases.reshape(-1)
    bias16 = jnp.broadcast_to(bias.astype(jnp.float32), (L,))
    return _mf(users.astype(jnp.int32), items.astype(jnp.int32),
               user_embeddings, item_embeddings, ub, ib, bias16)


# speedup vs baseline: 5.3427x; 5.3427x over previous
import functools

import jax
import jax.numpy as jnp
from jax import lax
from jax.experimental import pallas as pl
from jax.experimental.pallas import tpu as pltpu
from jax.experimental.pallas import tpu_sc as plsc

N_FACT = 32
B = 4096
NC = 2
NS = 16
NW = NC * NS
BPW = B // NW
L = 16
GROUPS = BPW // L
CHUNK = 4
NCH = BPW // CHUNK
TILE_W = 128


def _mf_body(users_h, items_h, uet_h, iet_h, ub_h, ib_h, bias_h, out_h,
             idx_u, idx_i, ue_slabs, ie_slabs, bu_v, bi_v, bias_v, out_v,
             sem_b, sem0, sem1):
    wid = lax.axis_index("s") * NC + lax.axis_index("c")
    base = wid * BPW

    pltpu.sync_copy(users_h.at[pl.ds(base, BPW)], idx_u)
    pltpu.sync_copy(items_h.at[pl.ds(base, BPW)], idx_i)
    pltpu.sync_copy(bias_h, bias_v)

    bias_cps = [
        pltpu.async_copy(ub_h.at[idx_u], bu_v, sem_b),
        pltpu.async_copy(ib_h.at[idx_i], bi_v, sem_b),
    ]

    lane = lax.broadcasted_iota(jnp.int32, (L,), 0)
    lane_hi = lane + L
    lane_masks = [lane == j for j in range(L)]
    sems = [sem0, sem1]

    vu = [idx_u[pl.ds(g * L, L)] for g in range(GROUPS)]
    vi = [idx_i[pl.ds(g * L, L)] for g in range(GROUPS)]

    def col_scalar(vec, j):
        return vec[j]

    def issue(k):
        par = k % 2
        cps = []
        for s in range(CHUNK):
            b = k * CHUNK + s
            g, j = b // L, b % L
            cu = col_scalar(vu[g], j)
            ci = col_scalar(vi[g], j)
            cu0 = pl.multiple_of((cu >> 7) << 7, TILE_W)
            ci0 = pl.multiple_of((ci >> 7) << 7, TILE_W)
            cps.append(pltpu.async_copy(
                uet_h.at[:, pl.ds(cu0, TILE_W)], ue_slabs.at[par, s],
                sems[par]))
            cps.append(pltpu.async_copy(
                iet_h.at[:, pl.ds(ci0, TILE_W)], ie_slabs.at[par, s],
                sems[par]))
        return cps

    acc = jnp.zeros((L,), jnp.float32)
    pending = issue(0)
    for k in range(NCH):
        nxt = issue(k + 1) if k + 1 < NCH else []
        for cp in pending:
            cp.wait()
        pending = nxt
        par = k % 2
        for s in range(CHUNK):
            b = k * CHUNK + s
            g, j = b // L, b % L
            cu = col_scalar(vu[g], j)
            ci = col_scalar(vi[g], j)
            ju = jnp.full((L,), cu & (TILE_W - 1), jnp.int32)
            ji = jnp.full((L,), ci & (TILE_W - 1), jnp.int32)
            u0 = plsc.load_gather(ue_slabs.at[par, s], [lane, ju])
            u1 = plsc.load_gather(ue_slabs.at[par, s], [lane_hi, ju])
            i0 = plsc.load_gather(ie_slabs.at[par, s], [lane, ji])
            i1 = plsc.load_gather(ie_slabs.at[par, s], [lane_hi, ji])
            dot = jnp.sum(u0 * i0 + u1 * i1)
            acc = jnp.where(lane_masks[j], jnp.full((L,), dot), acc)
            if j == L - 1:
                r0 = g * L
                if g == 0:
                    for cp in bias_cps:
                        cp.wait()
                    bias_vec = bias_v[...]
                res = acc + bu_v[pl.ds(r0, L)] + bi_v[pl.ds(r0, L)]
                res = res + bias_vec
                res = jnp.minimum(jnp.maximum(res, 1.0), 5.0)
                out_v[pl.ds(r0, L)] = res

    pltpu.sync_copy(out_v, out_h.at[pl.ds(base, BPW)])


_mf = pl.kernel(
    _mf_body,
    out_type=jax.ShapeDtypeStruct((B,), jnp.float32),
    mesh=plsc.VectorSubcoreMesh(core_axis_name="c", subcore_axis_name="s"),
    compiler_params=pltpu.CompilerParams(needs_layout_passes=False),
    scratch_types=[
        pltpu.VMEM((BPW,), jnp.int32),
        pltpu.VMEM((BPW,), jnp.int32),
        pltpu.VMEM((2, CHUNK, N_FACT, TILE_W), jnp.float32),
        pltpu.VMEM((2, CHUNK, N_FACT, TILE_W), jnp.float32),
        pltpu.VMEM((BPW,), jnp.float32),
        pltpu.VMEM((BPW,), jnp.float32),
        pltpu.VMEM((L,), jnp.float32),
        pltpu.VMEM((BPW,), jnp.float32),
        pltpu.SemaphoreType.DMA,
        pltpu.SemaphoreType.DMA,
        pltpu.SemaphoreType.DMA,
    ],
)


def kernel(users, items, user_embeddings, item_embeddings, user_biases,
           item_biases, bias):
    uet = user_embeddings.T
    iet = item_embeddings.T
    ub = user_biases.reshape(-1)
    ib = item_biases.reshape(-1)
    bias16 = jnp.broadcast_to(bias.astype(jnp.float32), (L,))
    return _mf(users.astype(jnp.int32), items.astype(jnp.int32),
               uet, iet, ub, ib, bias16)


# speedup vs baseline: 9.1943x; 1.7209x over previous
import functools

import jax
import jax.numpy as jnp
from jax import lax
from jax.experimental import pallas as pl
from jax.experimental.pallas import tpu as pltpu
from jax.experimental.pallas import tpu_sc as plsc

N_FACT = 32
N_ROWS = 1000000
B = 4096
NC = 2
NS = 16
NW = NC * NS
BPW = B // NW
L = 16
GROUPS = BPW // L
CHUNK = 4
NCH = BPW // CHUNK
TILE_W = 128
BIAS_ROWS = 7816
BIAS_PAD = BIAS_ROWS * TILE_W - N_ROWS


def _mf_body(users_h, items_h, uet_h, iet_h, ub_h, ib_h, bias_h, out_h,
             idx_u, idx_i, rid_u, rid_i, ue_slabs, ie_slabs,
             bu_rows, bi_rows, bias_v, out_v, sem_b, sem0, sem1):
    wid = lax.axis_index("s") * NC + lax.axis_index("c")
    base = wid * BPW

    pltpu.sync_copy(users_h.at[pl.ds(base, BPW)], idx_u)
    pltpu.sync_copy(items_h.at[pl.ds(base, BPW)], idx_i)
    pltpu.sync_copy(bias_h, bias_v)

    vu = [idx_u[pl.ds(g * L, L)] for g in range(GROUPS)]
    vi = [idx_i[pl.ds(g * L, L)] for g in range(GROUPS)]

    for g in range(GROUPS):
        rid_u[pl.ds(g * L, L)] = vu[g] >> 7
        rid_i[pl.ds(g * L, L)] = vi[g] >> 7

    bias_cps = [
        pltpu.async_copy(ub_h.at[rid_u], bu_rows, sem_b),
        pltpu.async_copy(ib_h.at[rid_i], bi_rows, sem_b),
    ]

    lane = lax.broadcasted_iota(jnp.int32, (L,), 0)
    lane_hi = lane + L
    lane_masks = [lane == j for j in range(L)]
    sems = [sem0, sem1]

    def issue(k):
        par = k % 2
        cps = []
        for s in range(CHUNK):
            b = k * CHUNK + s
            g, j = b // L, b % L
            cu = vu[g][j]
            ci = vi[g][j]
            cu0 = pl.multiple_of((cu >> 7) << 7, TILE_W)
            ci0 = pl.multiple_of((ci >> 7) << 7, TILE_W)
            cps.append(pltpu.async_copy(
                uet_h.at[:, pl.ds(cu0, TILE_W)], ue_slabs.at[par, s],
                sems[par]))
            cps.append(pltpu.async_copy(
                iet_h.at[:, pl.ds(ci0, TILE_W)], ie_slabs.at[par, s],
                sems[par]))
        return cps

    acc = jnp.zeros((L,), jnp.float32)
    pending = issue(0)
    for k in range(NCH):
        nxt = issue(k + 1) if k + 1 < NCH else []
        for cp in pending:
            cp.wait()
        pending = nxt
        par = k % 2
        for s in range(CHUNK):
            b = k * CHUNK + s
            g, j = b // L, b % L
            ju = jnp.full((L,), vu[g][j] & (TILE_W - 1), jnp.int32)
            ji = jnp.full((L,), vi[g][j] & (TILE_W - 1), jnp.int32)
            u0 = plsc.load_gather(ue_slabs.at[par, s], [lane, ju])
            u1 = plsc.load_gather(ue_slabs.at[par, s], [lane_hi, ju])
            i0 = plsc.load_gather(ie_slabs.at[par, s], [lane, ji])
            i1 = plsc.load_gather(ie_slabs.at[par, s], [lane_hi, ji])
            dot = jnp.sum(u0 * i0 + u1 * i1)
            acc = jnp.where(lane_masks[j], jnp.full((L,), dot), acc)
            if j == L - 1:
                r0 = g * L
                if g == 0:
                    for cp in bias_cps:
                        cp.wait()
                    bias_vec = bias_v[...]
                bu = plsc.load_gather(bu_rows, [lane + r0, vu[g] & (TILE_W - 1)])
                bi = plsc.load_gather(bi_rows, [lane + r0, vi[g] & (TILE_W - 1)])
                res = acc + bu + bi + bias_vec
                res = jnp.minimum(jnp.maximum(res, 1.0), 5.0)
                out_v[pl.ds(r0, L)] = res

    pltpu.sync_copy(out_v, out_h.at[pl.ds(base, BPW)])


_mf = pl.kernel(
    _mf_body,
    out_type=jax.ShapeDtypeStruct((B,), jnp.float32),
    mesh=plsc.VectorSubcoreMesh(core_axis_name="c", subcore_axis_name="s"),
    compiler_params=pltpu.CompilerParams(needs_layout_passes=False),
    scratch_types=[
        pltpu.VMEM((BPW,), jnp.int32),
        pltpu.VMEM((BPW,), jnp.int32),
        pltpu.VMEM((BPW,), jnp.int32),
        pltpu.VMEM((BPW,), jnp.int32),
        pltpu.VMEM((2, CHUNK, N_FACT, TILE_W), jnp.float32),
        pltpu.VMEM((2, CHUNK, N_FACT, TILE_W), jnp.float32),
        pltpu.VMEM((BPW, TILE_W), jnp.float32),
        pltpu.VMEM((BPW, TILE_W), jnp.float32),
        pltpu.VMEM((L,), jnp.float32),
        pltpu.VMEM((BPW,), jnp.float32),
        pltpu.SemaphoreType.DMA,
        pltpu.SemaphoreType.DMA,
        pltpu.SemaphoreType.DMA,
    ],
)


def _pad_bias(b2d):
    flat_pad = jnp.zeros((BIAS_PAD, 1), jnp.float32)
    return jnp.concatenate([b2d, flat_pad], axis=0).reshape(BIAS_ROWS, TILE_W)


def kernel(users, items, user_embeddings, item_embeddings, user_biases,
           item_biases, bias):
    uet = user_embeddings.T
    iet = item_embeddings.T
    ubp = _pad_bias(user_biases)
    ibp = _pad_bias(item_biases)
    bias16 = jnp.broadcast_to(bias.astype(jnp.float32), (L,))
    return _mf(users.astype(jnp.int32), items.astype(jnp.int32),
               uet, iet, ubp, ibp, bias16)


# speedup vs baseline: 9.6891x; 1.0538x over previous
import functools

import jax
import jax.numpy as jnp
from jax import lax
from jax.experimental import pallas as pl
from jax.experimental.pallas import tpu as pltpu
from jax.experimental.pallas import tpu_sc as plsc

N_FACT = 32
N_ROWS = 1000000
B = 4096
NC = 2
NS = 16
NW = NC * NS
BPW = B // NW
L = 16
GROUPS = BPW // L
CHUNK = 4
NCH = BPW // CHUNK
TILE_W = 128
DEPTH = 3
BIAS_ROWS = 7816
BIAS_PAD = BIAS_ROWS * TILE_W - N_ROWS


def _mf_body(users_h, items_h, uet_h, iet_h, ub_h, ib_h, bias_h, out_h,
             idx_u, idx_i, ue_slabs, ie_slabs,
             bu_v, bi_v, bias_v, out_v, sem_b, sem0, sem1, sem2):
    wid = lax.axis_index("s") * NC + lax.axis_index("c")
    base = wid * BPW

    pltpu.sync_copy(users_h.at[pl.ds(base, BPW)], idx_u)
    pltpu.sync_copy(items_h.at[pl.ds(base, BPW)], idx_i)
    pltpu.sync_copy(bias_h, bias_v)

    vu = [idx_u[pl.ds(g * L, L)] for g in range(GROUPS)]
    vi = [idx_i[pl.ds(g * L, L)] for g in range(GROUPS)]

    bias_cps = [
        pltpu.async_copy(ub_h.at[idx_u], bu_v, sem_b),
        pltpu.async_copy(ib_h.at[idx_i], bi_v, sem_b),
    ]

    lane = lax.broadcasted_iota(jnp.int32, (L,), 0)
    lane_hi = lane + L
    lane_masks = [lane == j for j in range(L)]
    sems = [sem0, sem1, sem2]

    def issue(k):
        par = k % DEPTH
        cps = []
        for s in range(CHUNK):
            b = k * CHUNK + s
            g, j = b // L, b % L
            cu = vu[g][j]
            ci = vi[g][j]
            cu0 = pl.multiple_of((cu >> 7) << 7, TILE_W)
            ci0 = pl.multiple_of((ci >> 7) << 7, TILE_W)
            cps.append(pltpu.async_copy(
                uet_h.at[:, pl.ds(cu0, TILE_W)], ue_slabs.at[par, s],
                sems[par]))
            cps.append(pltpu.async_copy(
                iet_h.at[:, pl.ds(ci0, TILE_W)], ie_slabs.at[par, s],
                sems[par]))
        return cps

    acc = jnp.zeros((L,), jnp.float32)
    inflight = [issue(k) for k in range(DEPTH - 1)]
    for k in range(NCH):
        if k + DEPTH - 1 < NCH:
            inflight.append(issue(k + DEPTH - 1))
        for cp in inflight.pop(0):
            cp.wait()
        par = k % DEPTH
        for s in range(CHUNK):
            b = k * CHUNK + s
            g, j = b // L, b % L
            ju = jnp.full((L,), vu[g][j] & (TILE_W - 1), jnp.int32)
            ji = jnp.full((L,), vi[g][j] & (TILE_W - 1), jnp.int32)
            u0 = plsc.load_gather(ue_slabs.at[par, s], [lane, ju])
            u1 = plsc.load_gather(ue_slabs.at[par, s], [lane_hi, ju])
            i0 = plsc.load_gather(ie_slabs.at[par, s], [lane, ji])
            i1 = plsc.load_gather(ie_slabs.at[par, s], [lane_hi, ji])
            dot = jnp.sum(u0 * i0 + u1 * i1)
            acc = jnp.where(lane_masks[j], jnp.full((L,), dot), acc)
            if j == L - 1:
                r0 = g * L
                if g == 0:
                    for cp in bias_cps:
                        cp.wait()
                    bias_vec = bias_v[...]
                res = acc + bu_v[pl.ds(r0, L)] + bi_v[pl.ds(r0, L)] + bias_vec
                res = jnp.minimum(jnp.maximum(res, 1.0), 5.0)
                out_v[pl.ds(r0, L)] = res

    pltpu.sync_copy(out_v, out_h.at[pl.ds(base, BPW)])


_mf = pl.kernel(
    _mf_body,
    out_type=jax.ShapeDtypeStruct((B,), jnp.float32),
    mesh=plsc.VectorSubcoreMesh(core_axis_name="c", subcore_axis_name="s"),
    compiler_params=pltpu.CompilerParams(needs_layout_passes=False),
    scratch_types=[
        pltpu.VMEM((BPW,), jnp.int32),
        pltpu.VMEM((BPW,), jnp.int32),
        pltpu.VMEM((DEPTH, CHUNK, N_FACT, TILE_W), jnp.float32),
        pltpu.VMEM((DEPTH, CHUNK, N_FACT, TILE_W), jnp.float32),
        pltpu.VMEM((BPW,), jnp.float32),
        pltpu.VMEM((BPW,), jnp.float32),
        pltpu.VMEM((L,), jnp.float32),
        pltpu.VMEM((BPW,), jnp.float32),
        pltpu.SemaphoreType.DMA,
        pltpu.SemaphoreType.DMA,
        pltpu.SemaphoreType.DMA,
        pltpu.SemaphoreType.DMA,
    ],
)


def _pad_bias(b2d):
    flat_pad = jnp.zeros((BIAS_PAD, 1), jnp.float32)
    return jnp.concatenate([b2d, flat_pad], axis=0).reshape(-1)


def kernel(users, items, user_embeddings, item_embeddings, user_biases,
           item_biases, bias):
    uet = user_embeddings.T
    iet = item_embeddings.T
    ubp = _pad_bias(user_biases)
    ibp = _pad_bias(item_biases)
    bias16 = jnp.broadcast_to(bias.astype(jnp.float32), (L,))
    return _mf(users.astype(jnp.int32), items.astype(jnp.int32),
               uet, iet, ubp, ibp, bias16)


# speedup vs baseline: 10.3021x; 1.0633x over previous
import functools

import jax
import jax.numpy as jnp
from jax import lax
from jax.experimental import pallas as pl
from jax.experimental.pallas import tpu as pltpu
from jax.experimental.pallas import tpu_sc as plsc

N_FACT = 32
N_ROWS = 1000000
B = 4096
NC = 2
NS = 16
NW = NC * NS
BPW = B // NW
L = 16
GROUPS = BPW // L
CHUNK = 2
NCH = BPW // CHUNK
TILE_W = 128
DEPTH = 6
BIAS_ROWS = 7816
BIAS_PAD = BIAS_ROWS * TILE_W - N_ROWS


def _mf_body(users_h, items_h, uet_h, iet_h, ub_h, ib_h, bias_h, out_h,
             idx_u, idx_i, ue_slabs, ie_slabs,
             bu_v, bi_v, bias_v, out_v, sem_b, sem0, sem1, sem2, sem3, sem4,
             sem5):
    wid = lax.axis_index("s") * NC + lax.axis_index("c")
    base = wid * BPW

    pltpu.sync_copy(users_h.at[pl.ds(base, BPW)], idx_u)
    pltpu.sync_copy(items_h.at[pl.ds(base, BPW)], idx_i)
    pltpu.sync_copy(bias_h, bias_v)

    vu = [idx_u[pl.ds(g * L, L)] for g in range(GROUPS)]
    vi = [idx_i[pl.ds(g * L, L)] for g in range(GROUPS)]

    bias_cps = [
        pltpu.async_copy(ub_h.at[idx_u], bu_v, sem_b),
        pltpu.async_copy(ib_h.at[idx_i], bi_v, sem_b),
    ]

    lane = lax.broadcasted_iota(jnp.int32, (L,), 0)
    lane_hi = lane + L
    lane_masks = [lane == j for j in range(L)]
    sems = [sem0, sem1, sem2, sem3, sem4, sem5]

    def issue(k):
        par = k % DEPTH
        cps = []
        for s in range(CHUNK):
            b = k * CHUNK + s
            g, j = b // L, b % L
            cu = vu[g][j]
            ci = vi[g][j]
            cu0 = pl.multiple_of((cu >> 7) << 7, TILE_W)
            ci0 = pl.multiple_of((ci >> 7) << 7, TILE_W)
            cps.append(pltpu.async_copy(
                uet_h.at[:, pl.ds(cu0, TILE_W)], ue_slabs.at[par, s],
                sems[par]))
            cps.append(pltpu.async_copy(
                iet_h.at[:, pl.ds(ci0, TILE_W)], ie_slabs.at[par, s],
                sems[par]))
        return cps

    acc = jnp.zeros((L,), jnp.float32)
    inflight = [issue(k) for k in range(DEPTH - 1)]
    for k in range(NCH):
        if k + DEPTH - 1 < NCH:
            inflight.append(issue(k + DEPTH - 1))
        for cp in inflight.pop(0):
            cp.wait()
        par = k % DEPTH
        for s in range(CHUNK):
            b = k * CHUNK + s
            g, j = b // L, b % L
            ju = jnp.full((L,), vu[g][j] & (TILE_W - 1), jnp.int32)
            ji = jnp.full((L,), vi[g][j] & (TILE_W - 1), jnp.int32)
            u0 = plsc.load_gather(ue_slabs.at[par, s], [lane, ju])
            u1 = plsc.load_gather(ue_slabs.at[par, s], [lane_hi, ju])
            i0 = plsc.load_gather(ie_slabs.at[par, s], [lane, ji])
            i1 = plsc.load_gather(ie_slabs.at[par, s], [lane_hi, ji])
            dot = jnp.sum(u0 * i0 + u1 * i1)
            acc = jnp.where(lane_masks[j], jnp.full((L,), dot), acc)
            if j == L - 1:
                r0 = g * L
                if g == 0:
                    for cp in bias_cps:
                        cp.wait()
                    bias_vec = bias_v[...]
                res = acc + bu_v[pl.ds(r0, L)] + bi_v[pl.ds(r0, L)] + bias_vec
                res = jnp.minimum(jnp.maximum(res, 1.0), 5.0)
                out_v[pl.ds(r0, L)] = res

    pltpu.sync_copy(out_v, out_h.at[pl.ds(base, BPW)])


_mf = pl.kernel(
    _mf_body,
    out_type=jax.ShapeDtypeStruct((B,), jnp.float32),
    mesh=plsc.VectorSubcoreMesh(core_axis_name="c", subcore_axis_name="s"),
    compiler_params=pltpu.CompilerParams(needs_layout_passes=False),
    scratch_types=[
        pltpu.VMEM((BPW,), jnp.int32),
        pltpu.VMEM((BPW,), jnp.int32),
        pltpu.VMEM((DEPTH, CHUNK, N_FACT, TILE_W), jnp.float32),
        pltpu.VMEM((DEPTH, CHUNK, N_FACT, TILE_W), jnp.float32),
        pltpu.VMEM((BPW,), jnp.float32),
        pltpu.VMEM((BPW,), jnp.float32),
        pltpu.VMEM((L,), jnp.float32),
        pltpu.VMEM((BPW,), jnp.float32),
        pltpu.SemaphoreType.DMA,
        pltpu.SemaphoreType.DMA,
        pltpu.SemaphoreType.DMA,
        pltpu.SemaphoreType.DMA,
        pltpu.SemaphoreType.DMA,
        pltpu.SemaphoreType.DMA,
        pltpu.SemaphoreType.DMA,
    ],
)


def _pad_bias(b2d):
    flat_pad = jnp.zeros((BIAS_PAD, 1), jnp.float32)
    return jnp.concatenate([b2d, flat_pad], axis=0).reshape(-1)


def kernel(users, items, user_embeddings, item_embeddings, user_biases,
           item_biases, bias):
    uet = user_embeddings.T
    iet = item_embeddings.T
    ubp = _pad_bias(user_biases)
    ibp = _pad_bias(item_biases)
    bias16 = jnp.broadcast_to(bias.astype(jnp.float32), (L,))
    return _mf(users.astype(jnp.int32), items.astype(jnp.int32),
               uet, iet, ubp, ibp, bias16)


# speedup vs baseline: 11.1577x; 1.0830x over previous
import functools

import jax
import jax.numpy as jnp
from jax import lax
from jax.experimental import pallas as pl
from jax.experimental.pallas import tpu as pltpu
from jax.experimental.pallas import tpu_sc as plsc

N_FACT = 32
N_ROWS = 1000000
B = 4096
NC = 2
NS = 16
NW = NC * NS
BPW = B // NW
L = 16
GROUPS = BPW // L
CHUNK = 2
NCH = BPW // CHUNK
TILE_W = 128
DEPTH = 6
BIAS_ROWS = 7816
BIAS_PAD = BIAS_ROWS * TILE_W - N_ROWS


def _main_body(users_h, items_h, uet_h, iet_h, out_h,
               idx_u, idx_i, ue_slabs, ie_slabs, out_v,
               sem0, sem1, sem2, sem3, sem4, sem5):
    wid = lax.axis_index("s") * NC + lax.axis_index("c")
    base = wid * BPW

    pltpu.sync_copy(users_h.at[pl.ds(base, BPW)], idx_u)
    pltpu.sync_copy(items_h.at[pl.ds(base, BPW)], idx_i)

    vu = [idx_u[pl.ds(g * L, L)] for g in range(GROUPS)]
    vi = [idx_i[pl.ds(g * L, L)] for g in range(GROUPS)]

    lane = lax.broadcasted_iota(jnp.int32, (L,), 0)
    lane_hi = lane + L
    lane_masks = [lane == j for j in range(L)]
    sems = [sem0, sem1, sem2, sem3, sem4, sem5]

    def issue(k):
        par = k % DEPTH
        cps = []
        for s in range(CHUNK):
            b = k * CHUNK + s
            g, j = b // L, b % L
            cu = vu[g][j]
            ci = vi[g][j]
            cu0 = pl.multiple_of((cu >> 7) << 7, TILE_W)
            ci0 = pl.multiple_of((ci >> 7) << 7, TILE_W)
            cps.append(pltpu.async_copy(
                uet_h.at[:, pl.ds(cu0, TILE_W)], ue_slabs.at[par, s],
                sems[par]))
            cps.append(pltpu.async_copy(
                iet_h.at[:, pl.ds(ci0, TILE_W)], ie_slabs.at[par, s],
                sems[par]))
        return cps

    acc = jnp.zeros((L,), jnp.float32)
    inflight = [issue(k) for k in range(DEPTH - 1)]
    for k in range(NCH):
        if k + DEPTH - 1 < NCH:
            inflight.append(issue(k + DEPTH - 1))
        for cp in inflight.pop(0):
            cp.wait()
        par = k % DEPTH
        for s in range(CHUNK):
            b = k * CHUNK + s
            g, j = b // L, b % L
            ju = jnp.full((L,), vu[g][j] & (TILE_W - 1), jnp.int32)
            ji = jnp.full((L,), vi[g][j] & (TILE_W - 1), jnp.int32)
            u0 = plsc.load_gather(ue_slabs.at[par, s], [lane, ju])
            u1 = plsc.load_gather(ue_slabs.at[par, s], [lane_hi, ju])
            i0 = plsc.load_gather(ie_slabs.at[par, s], [lane, ji])
            i1 = plsc.load_gather(ie_slabs.at[par, s], [lane_hi, ji])
            dot = jnp.sum(u0 * i0 + u1 * i1)
            acc = jnp.where(lane_masks[j], jnp.full((L,), dot), acc)
            if j == L - 1:
                out_v[pl.ds(g * L, L)] = acc

    pltpu.sync_copy(out_v, out_h.at[pl.ds(base, BPW)])


_mf_main = pl.kernel(
    _main_body,
    out_type=jax.ShapeDtypeStruct((B,), jnp.float32),
    mesh=plsc.VectorSubcoreMesh(core_axis_name="c", subcore_axis_name="s"),
    compiler_params=pltpu.CompilerParams(needs_layout_passes=False),
    scratch_types=[
        pltpu.VMEM((BPW,), jnp.int32),
        pltpu.VMEM((BPW,), jnp.int32),
        pltpu.VMEM((DEPTH, CHUNK, N_FACT, TILE_W), jnp.float32),
        pltpu.VMEM((DEPTH, CHUNK, N_FACT, TILE_W), jnp.float32),
        pltpu.VMEM((BPW,), jnp.float32),
        pltpu.SemaphoreType.DMA,
        pltpu.SemaphoreType.DMA,
        pltpu.SemaphoreType.DMA,
        pltpu.SemaphoreType.DMA,
        pltpu.SemaphoreType.DMA,
        pltpu.SemaphoreType.DMA,
    ],
)


def _epi_body(users_h, items_h, ub_h, ib_h, bias_h, dots_h, out_h,
              idx_u, idx_i, bu_v, bi_v, bias_v, dots_v, out_v, sem_b):
    wid = lax.axis_index("s") * NC + lax.axis_index("c")
    base = wid * BPW

    pltpu.sync_copy(users_h.at[pl.ds(base, BPW)], idx_u)
    pltpu.sync_copy(items_h.at[pl.ds(base, BPW)], idx_i)
    cps = [
        pltpu.async_copy(ub_h.at[idx_u], bu_v, sem_b),
        pltpu.async_copy(ib_h.at[idx_i], bi_v, sem_b),
    ]
    pltpu.sync_copy(dots_h.at[pl.ds(base, BPW)], dots_v)
    pltpu.sync_copy(bias_h, bias_v)
    for cp in cps:
        cp.wait()
    bias_vec = bias_v[...]
    for g in range(GROUPS):
        r0 = g * L
        res = dots_v[pl.ds(r0, L)] + bu_v[pl.ds(r0, L)] + bi_v[pl.ds(r0, L)]
        res = res + bias_vec
        res = jnp.minimum(jnp.maximum(res, 1.0), 5.0)
        out_v[pl.ds(r0, L)] = res
    pltpu.sync_copy(out_v, out_h.at[pl.ds(base, BPW)])


_mf_epi = pl.kernel(
    _epi_body,
    out_type=jax.ShapeDtypeStruct((B,), jnp.float32),
    mesh=plsc.VectorSubcoreMesh(core_axis_name="c", subcore_axis_name="s"),
    compiler_params=pltpu.CompilerParams(needs_layout_passes=False),
    scratch_types=[
        pltpu.VMEM((BPW,), jnp.int32),
        pltpu.VMEM((BPW,), jnp.int32),
        pltpu.VMEM((BPW,), jnp.float32),
        pltpu.VMEM((BPW,), jnp.float32),
        pltpu.VMEM((L,), jnp.float32),
        pltpu.VMEM((BPW,), jnp.float32),
        pltpu.VMEM((BPW,), jnp.float32),
        pltpu.SemaphoreType.DMA,
    ],
)


def _pad_bias(b2d):
    flat_pad = jnp.zeros((BIAS_PAD, 1), jnp.float32)
    return jnp.concatenate([b2d, flat_pad], axis=0).reshape(-1)


def kernel(users, items, user_embeddings, item_embeddings, user_biases,
           item_biases, bias):
    u32 = users.astype(jnp.int32)
    i32 = items.astype(jnp.int32)
    uet = user_embeddings.T
    iet = item_embeddings.T
    ubp = _pad_bias(user_biases)
    ibp = _pad_bias(item_biases)
    bias16 = jnp.broadcast_to(bias.astype(jnp.float32), (L,))
    dots = _mf_main(u32, i32, uet, iet)
    return _mf_epi(u32, i32, ubp, ibp, bias16, dots)
